# Initial kernel scaffold; baseline (speedup 1.0000x reference)
import jax
import jax.numpy as jnp
from jax.experimental import pallas as pl


def kernel(node_fts, edge_fts, edges, Wn, We, a_node, a_edge):
    raise NotImplementedError("write your pallas kernel here")


# speedup vs baseline: 28.4922x; 28.4922x over previous
import functools

import jax
import jax.numpy as jnp
from jax import lax
from jax.experimental import pallas as pl
from jax.experimental.pallas import tpu as pltpu
from jax.experimental.pallas import tpu_sc as plsc

N = 10000
E = 320000
DIN = 128
DOUT = 32
EIN = 16
EOUT = 16
H = 4
ALPHA = 0.2
EPS = 1e-16

NC = 2
NS = 16
L = 16

C = 80
EPT = E // NS
NCHUNK = EPT // C
NP = 10240
ROWS_PT = NP // NS


def _matmul_body(x_ref, w_ref, o_ref):
    o_ref[...] = jnp.dot(x_ref[...], w_ref[...],
                         preferred_element_type=jnp.float32)


def _prep_node(node_fts, wbig):
    kcols = wbig.shape[1]
    return pl.pallas_call(
        _matmul_body,
        grid=(10,),
        in_specs=[
            pl.BlockSpec((N // 10, DIN), lambda i: (i, 0)),
            pl.BlockSpec((DIN, kcols), lambda i: (0, 0)),
        ],
        out_specs=pl.BlockSpec((N // 10, kcols), lambda i: (i, 0)),
        out_shape=jax.ShapeDtypeStruct((N, kcols), jnp.float32),
    )(node_fts, wbig)


def _prep_edge(edge_fts, wea):
    return pl.pallas_call(
        _matmul_body,
        grid=(64,),
        in_specs=[
            pl.BlockSpec((E // 64, EIN), lambda i: (i, 0)),
            pl.BlockSpec((EIN, H), lambda i: (0, 0)),
        ],
        out_specs=pl.BlockSpec((E // 64, H), lambda i: (i, 0)),
        out_shape=jax.ShapeDtypeStruct((E, H), jnp.float32),
    )(edge_fts, wea)


def _sc_body(hh_hbm, tab_hbm, src_hbm, dst_hbm, ge_hbm, ef_hbm,
             zn_hbm, ze_hbm, zs_hbm,
             outn_hbm, oute_hbm, outs_hbm,
             shn, she, shs, shtab,
             src_v, dst_v, idx_v, tdr_v, tsr_v, ge_v, hh_v, ef_v,
             nrow_v, erow_v, srow_v,
             gsem, dsem, ssem):
    cid = lax.axis_index("c")
    sid = lax.axis_index("s")
    coff = cid * N

    @pl.when(sid == 0)
    def _load_tables():
        pltpu.sync_copy(tab_hbm.at[pl.ds(coff, N)], shtab)

    r0 = sid * ROWS_PT
    pltpu.sync_copy(zn_hbm, shn.at[pl.ds(r0, ROWS_PT)])
    pltpu.sync_copy(ze_hbm, she.at[pl.ds(r0, ROWS_PT)])
    pltpu.sync_copy(zs_hbm, shs.at[pl.ds(r0, ROWS_PT)])
    plsc.subcore_barrier()

    base0 = sid * EPT
    lanes = lax.iota(jnp.int32, 16)

    def chunk_body(k, carry):
        base = base0 + k * C
        pltpu.sync_copy(src_hbm.at[pl.ds(base, C)], src_v)
        pltpu.sync_copy(dst_hbm.at[pl.ds(base, C)], dst_v)

        for g in range(C // L):
            idx_v[pl.ds(g * L, L)] = src_v[pl.ds(g * L, L)] + coff
        gather = pltpu.async_copy(hh_hbm.at[idx_v], hh_v, gsem)
        gat_d = pltpu.async_copy(shtab.at[dst_v], tdr_v, dsem)
        gat_s = pltpu.async_copy(shtab.at[src_v], tsr_v, ssem)
        pltpu.sync_copy(ge_hbm.at[pl.ds(base, C)], ge_v)
        pltpu.sync_copy(ef_hbm.at[pl.ds(base, C)], ef_v)
        gat_d.wait()
        gat_s.wait()

        scores = []
        for g in range(C // L):
            row_i = lanes + (g * L)
            svecs = []
            for j in range(2):
                c0 = jnp.full((16,), 2 * j, jnp.int32)
                sd = plsc.load_gather(tdr_v, [row_i, c0])
                esd = plsc.load_gather(tdr_v, [row_i, c0 + 1])
                ss = plsc.load_gather(tsr_v, [row_i, c0 + 4])
                ess = plsc.load_gather(tsr_v, [row_i, c0 + 5])
                gcol = jnp.full((16,), 2 * cid + j, jnp.int32)
                gej = plsc.load_gather(ge_v, [row_i, gcol])
                ns = sd + ss
                ns = jnp.where(ns > 0, ns, ALPHA * ns)
                en = jnp.exp(ns)
                es = esd + ess + gej
                es = jnp.where(es > 0, es, ALPHA * es)
                ee = jnp.exp(es)
                svecs += [en, ee]
                sc0 = jnp.full((16,), 4 * j, jnp.int32)
                plsc.store_scatter(srow_v, [row_i, sc0], en)
                plsc.store_scatter(srow_v, [row_i, sc0 + 1], en * en)
                plsc.store_scatter(srow_v, [row_i, sc0 + 2], ee)
                plsc.store_scatter(srow_v, [row_i, sc0 + 3], ee * ee)
            scores.append(svecs)

        gather.wait()

        for g in range(C // L):
            ena, eea, enb, eeb = scores[g]
            for ii in range(L):
                i = g * L + ii
                ea = ena[ii]
                eb = enb[ii]
                fa = eea[ii]
                fb = eeb[ii]
                for j in range(2):
                    nrow_v[i, pl.ds(j * L, L)] = (
                        ea * hh_v[i, pl.ds(j * L, L)])
                    nrow_v[i, pl.ds(DOUT + j * L, L)] = (
                        eb * hh_v[i, pl.ds(DOUT + j * L, L)])
                ef = ef_v[i, :]
                erow_v[i, pl.ds(0, L)] = fa * ef
                erow_v[i, pl.ds(L, L)] = fb * ef

        pltpu.sync_copy(srow_v, shs.at[dst_v], add=True)
        pltpu.sync_copy(nrow_v, shn.at[dst_v], add=True)
        pltpu.sync_copy(erow_v, she.at[dst_v], add=True)
        return carry

    lax.fori_loop(0, NCHUNK, chunk_body, 0)

    plsc.subcore_barrier()
    o0 = cid * NP + r0
    pltpu.sync_copy(shn.at[pl.ds(r0, ROWS_PT)], outn_hbm.at[pl.ds(o0, ROWS_PT)])
    pltpu.sync_copy(she.at[pl.ds(r0, ROWS_PT)], oute_hbm.at[pl.ds(o0, ROWS_PT)])
    pltpu.sync_copy(shs.at[pl.ds(r0, ROWS_PT)], outs_hbm.at[pl.ds(o0, ROWS_PT)])


def _sc_edge_pass(hh_cat, tab, src, dst, ge, edge_fts):
    mesh = plsc.VectorSubcoreMesh(core_axis_name="c", subcore_axis_name="s",
                                  num_cores=NC, num_subcores=NS)
    zn = jnp.zeros((ROWS_PT, 2 * DOUT), jnp.float32)
    ze = jnp.zeros((ROWS_PT, 2 * EOUT), jnp.float32)
    zs = jnp.zeros((ROWS_PT, 8), jnp.float32)
    run = pl.kernel(
        _sc_body,
        compiler_params=pltpu.CompilerParams(needs_layout_passes=False,
                                             use_tc_tiling_on_sc=False),
        out_type=[
            jax.ShapeDtypeStruct((NC * NP, 2 * DOUT), jnp.float32),
            jax.ShapeDtypeStruct((NC * NP, 2 * EOUT), jnp.float32),
            jax.ShapeDtypeStruct((NC * NP, 8), jnp.float32),
        ],
        mesh=mesh,
        scratch_types=[
            pltpu.VMEM_SHARED((NP, 2 * DOUT), jnp.float32),
            pltpu.VMEM_SHARED((NP, 2 * EOUT), jnp.float32),
            pltpu.VMEM_SHARED((NP, 8), jnp.float32),
            pltpu.VMEM_SHARED((N, 8), jnp.float32),
            pltpu.VMEM((C,), jnp.int32),
            pltpu.VMEM((C,), jnp.int32),
            pltpu.VMEM((C,), jnp.int32),
            pltpu.VMEM((C, 8), jnp.float32),
            pltpu.VMEM((C, 8), jnp.float32),
            pltpu.VMEM((C, H), jnp.float32),
            pltpu.VMEM((C, 2 * DOUT), jnp.float32),
            pltpu.VMEM((C, EIN), jnp.float32),
            pltpu.VMEM((C, 2 * DOUT), jnp.float32),
            pltpu.VMEM((C, 2 * EOUT), jnp.float32),
            pltpu.VMEM((C, 8), jnp.float32),
            pltpu.SemaphoreType.DMA,
            pltpu.SemaphoreType.DMA,
            pltpu.SemaphoreType.DMA,
        ],
    )
    return run(hh_cat, tab, src, dst, ge, edge_fts, zn, ze, zs)


def _weights_body(stat_ref, o_ref):
    s = stat_ref[...]
    nvars = []
    evars = []
    for c in range(NC):
        for j in range(2):
            den = s[c, :, 4 * j]
            q = s[c, :, 4 * j + 1]
            eden = s[c, :, 4 * j + 2]
            eq = s[c, :, 4 * j + 3]
            mean_n = jnp.sum(den / (den + EPS)) / E
            ex2_n = jnp.sum(q / (den + EPS) ** 2) / E
            mean_e = jnp.sum(eden / (eden + EPS)) / E
            ex2_e = jnp.sum(eq / (eden + EPS) ** 2) / E
            nvars.append(ex2_n - mean_n * mean_n)
            evars.append(ex2_e - mean_e * mean_e)
    nv = jnp.exp(jnp.clip(jnp.stack(nvars), -2.0, 2.0))
    nv = nv / jnp.sum(nv)
    ev = jnp.exp(jnp.clip(jnp.stack(evars), -2.0, 2.0))
    ev = ev / jnp.sum(ev)
    o_ref[...] = jnp.stack([nv, ev])[None]


def _head_weights(stat):
    return pl.pallas_call(
        _weights_body,
        out_shape=jax.ShapeDtypeStruct((1, NC, H), jnp.float32),
    )(stat)


def _finalize_body(accn_ref, acce_ref, stat_ref, w_ref, we_ref, o_ref):
    parts_n = []
    parts_e = []
    for c in range(NC):
        for j in range(2):
            h = 2 * c + j
            den = stat_ref[c, :, 4 * j]
            eden = stat_ref[c, :, 4 * j + 2]
            num = accn_ref[c, :, j * DOUT:(j + 1) * DOUT]
            enum = acce_ref[c, :, j * EOUT:(j + 1) * EOUT]
            nscale = w_ref[0, 0:1, h:h + 1]
            escale = w_ref[0, 1:2, h:h + 1]
            node_out = num / (den[:, None] + EPS) * nscale
            tmp = enum / (eden[:, None] + EPS)
            edge_out = jnp.dot(tmp, we_ref[h],
                               preferred_element_type=jnp.float32) * escale
            parts_n.append(node_out)
            parts_e.append(edge_out)
    o_ref[...] = jnp.concatenate(parts_n + parts_e, axis=1)


def _finalize(accn, acce, stat, w, we):
    nb = N // 10
    return pl.pallas_call(
        _finalize_body,
        grid=(10,),
        in_specs=[
            pl.BlockSpec((NC, nb, 2 * DOUT), lambda i: (0, i, 0)),
            pl.BlockSpec((NC, nb, 2 * EOUT), lambda i: (0, i, 0)),
            pl.BlockSpec((NC, nb, 8), lambda i: (0, i, 0)),
            pl.BlockSpec((1, NC, H), lambda i: (0, 0, 0)),
            pl.BlockSpec((H, EOUT, EOUT), lambda i: (0, 0, 0)),
        ],
        out_specs=pl.BlockSpec((nb, H * (DOUT + EOUT)), lambda i: (i, 0)),
        out_shape=jax.ShapeDtypeStruct((N, H * (DOUT + EOUT)), jnp.float32),
    )(accn, acce, stat, w, we)


def kernel(node_fts, edge_fts, edges, Wn, We, a_node, a_edge):
    w_sd = jnp.einsum('hdo,ho->dh', Wn, a_node[:, :DOUT])
    w_ss = jnp.einsum('hdo,ho->dh', Wn, a_node[:, DOUT:])
    w_esd = jnp.einsum('hdo,ho->dh', Wn, a_edge[:, :DOUT])
    w_ess = jnp.einsum('hdo,ho->dh', Wn, a_edge[:, DOUT:2 * DOUT])
    wea = jnp.einsum('heo,ho->eh', We, a_edge[:, 2 * DOUT:])

    wn_flat = jnp.concatenate([Wn[h] for h in range(H)], axis=1)
    tab_cols = jnp.stack(
        [w_sd[:, 0], w_esd[:, 0], w_sd[:, 1], w_esd[:, 1],
         w_ss[:, 0], w_ess[:, 0], w_ss[:, 1], w_ess[:, 1],
         w_sd[:, 2], w_esd[:, 2], w_sd[:, 3], w_esd[:, 3],
         w_ss[:, 2], w_ess[:, 2], w_ss[:, 3], w_ess[:, 3]], axis=1)
    wbig = jnp.concatenate([wn_flat, tab_cols], axis=1)

    big = _prep_node(node_fts, wbig)
    ge = _prep_edge(edge_fts, wea)

    hh_cat = jnp.concatenate([big[:, :64], big[:, 64:128]], axis=0)
    tab = jnp.concatenate([big[:, 128:136], big[:, 136:144]], axis=0)

    src = jnp.asarray(edges[:, 0], jnp.int32)
    dst = jnp.asarray(edges[:, 1], jnp.int32)

    outn, oute, outs = _sc_edge_pass(hh_cat, tab, src, dst, ge, edge_fts)

    accn = outn.reshape(NC, NP, 2 * DOUT)[:, :N]
    acce = oute.reshape(NC, NP, 2 * EOUT)[:, :N]
    stat = outs.reshape(NC, NP, 8)[:, :N]

    w = _head_weights(stat)
    return _finalize(accn, acce, stat, w, We)


# speedup vs baseline: 33.8029x; 1.1864x over previous
import functools

import jax
import jax.numpy as jnp
from jax import lax
from jax.experimental import pallas as pl
from jax.experimental.pallas import tpu as pltpu
from jax.experimental.pallas import tpu_sc as plsc

N = 10000
E = 320000
DIN = 128
DOUT = 32
EIN = 16
EOUT = 16
H = 4
ALPHA = 0.2
EPS = 1e-16

NC = 2
NS = 16
L = 16

C = 80
EPT = E // NS
NCHUNK = EPT // C
NP = 10240
ROWS_PT = NP // NS


def _matmul_body(x_ref, w_ref, o_ref):
    o_ref[...] = jnp.dot(x_ref[...], w_ref[...],
                         preferred_element_type=jnp.float32)


def _prep_node(node_fts, wbig):
    kcols = wbig.shape[1]
    return pl.pallas_call(
        _matmul_body,
        grid=(10,),
        in_specs=[
            pl.BlockSpec((N // 10, DIN), lambda i: (i, 0)),
            pl.BlockSpec((DIN, kcols), lambda i: (0, 0)),
        ],
        out_specs=pl.BlockSpec((N // 10, kcols), lambda i: (i, 0)),
        out_shape=jax.ShapeDtypeStruct((N, kcols), jnp.float32),
    )(node_fts, wbig)


def _prep_edge(edge_fts, wea):
    return pl.pallas_call(
        _matmul_body,
        grid=(64,),
        in_specs=[
            pl.BlockSpec((E // 64, EIN), lambda i: (i, 0)),
            pl.BlockSpec((EIN, H), lambda i: (0, 0)),
        ],
        out_specs=pl.BlockSpec((E // 64, H), lambda i: (i, 0)),
        out_shape=jax.ShapeDtypeStruct((E, H), jnp.float32),
    )(edge_fts, wea)


def _sc_body(hh_hbm, tab_hbm, src_hbm, dst_hbm, ge_hbm, ef_hbm,
             zn_hbm, ze_hbm, zs_hbm,
             outn_hbm, oute_hbm, outs_hbm,
             shn, she, shs, shtab,
             src_v, dst_v, idx_v, tdr_v, tsr_v, ge_v, hh_v, ef_v,
             nrow_v, erow_v, srow_v,
             gsem, dsem, ssem, lsem0, lsem1, lsem2, lsem3):
    cid = lax.axis_index("c")
    sid = lax.axis_index("s")
    coff = cid * N

    @pl.when(sid == 0)
    def _load_tables():
        pltpu.sync_copy(tab_hbm.at[pl.ds(coff, N)], shtab)

    r0 = sid * ROWS_PT
    pltpu.sync_copy(zn_hbm, shn.at[pl.ds(r0, ROWS_PT)])
    pltpu.sync_copy(ze_hbm, she.at[pl.ds(r0, ROWS_PT)])
    pltpu.sync_copy(zs_hbm, shs.at[pl.ds(r0, ROWS_PT)])
    plsc.subcore_barrier()

    base0 = sid * EPT
    lanes = lax.iota(jnp.int32, 16)

    def chunk_body(k, carry):
        base = base0 + k * C
        ld_src = pltpu.async_copy(src_hbm.at[pl.ds(base, C)], src_v, lsem0)
        ld_dst = pltpu.async_copy(dst_hbm.at[pl.ds(base, C)], dst_v, lsem1)
        ld_ge = pltpu.async_copy(ge_hbm.at[pl.ds(base, C)], ge_v, lsem2)
        ld_ef = pltpu.async_copy(ef_hbm.at[pl.ds(base, C)], ef_v, lsem3)
        ld_src.wait()
        ld_dst.wait()

        for g in range(C // L):
            idx_v[pl.ds(g * L, L)] = src_v[pl.ds(g * L, L)] + coff
        gather = pltpu.async_copy(hh_hbm.at[idx_v], hh_v, gsem)
        gat_d = pltpu.async_copy(shtab.at[dst_v], tdr_v, dsem)
        gat_s = pltpu.async_copy(shtab.at[src_v], tsr_v, ssem)
        ld_ge.wait()
        ld_ef.wait()
        gat_d.wait()
        gat_s.wait()

        scores = []
        for g in range(C // L):
            row_i = lanes + (g * L)
            svecs = []
            for j in range(2):
                c0 = jnp.full((16,), 2 * j, jnp.int32)
                sd = plsc.load_gather(tdr_v, [row_i, c0])
                esd = plsc.load_gather(tdr_v, [row_i, c0 + 1])
                ss = plsc.load_gather(tsr_v, [row_i, c0 + 4])
                ess = plsc.load_gather(tsr_v, [row_i, c0 + 5])
                gcol = jnp.full((16,), 2 * cid + j, jnp.int32)
                gej = plsc.load_gather(ge_v, [row_i, gcol])
                ns = sd + ss
                ns = jnp.where(ns > 0, ns, ALPHA * ns)
                en = jnp.exp(ns)
                es = esd + ess + gej
                es = jnp.where(es > 0, es, ALPHA * es)
                ee = jnp.exp(es)
                svecs += [en, ee]
                sc0 = jnp.full((16,), 4 * j, jnp.int32)
                plsc.store_scatter(srow_v, [row_i, sc0], en)
                plsc.store_scatter(srow_v, [row_i, sc0 + 1], en * en)
                plsc.store_scatter(srow_v, [row_i, sc0 + 2], ee)
                plsc.store_scatter(srow_v, [row_i, sc0 + 3], ee * ee)
            scores.append(svecs)

        gather.wait()

        for g in range(C // L):
            ena, eea, enb, eeb = scores[g]
            for ii in range(L):
                i = g * L + ii
                ea = ena[ii]
                eb = enb[ii]
                fa = eea[ii]
                fb = eeb[ii]
                for j in range(2):
                    nrow_v[i, pl.ds(j * L, L)] = (
                        ea * hh_v[i, pl.ds(j * L, L)])
                    nrow_v[i, pl.ds(DOUT + j * L, L)] = (
                        eb * hh_v[i, pl.ds(DOUT + j * L, L)])
                ef = ef_v[i, :]
                erow_v[i, pl.ds(0, L)] = fa * ef
                erow_v[i, pl.ds(L, L)] = fb * ef

        pltpu.sync_copy(srow_v, shs.at[dst_v], add=True)
        pltpu.sync_copy(nrow_v, shn.at[dst_v], add=True)
        pltpu.sync_copy(erow_v, she.at[dst_v], add=True)
        return carry

    lax.fori_loop(0, NCHUNK, chunk_body, 0)

    plsc.subcore_barrier()
    o0 = cid * NP + r0
    pltpu.sync_copy(shn.at[pl.ds(r0, ROWS_PT)], outn_hbm.at[pl.ds(o0, ROWS_PT)])
    pltpu.sync_copy(she.at[pl.ds(r0, ROWS_PT)], oute_hbm.at[pl.ds(o0, ROWS_PT)])
    pltpu.sync_copy(shs.at[pl.ds(r0, ROWS_PT)], outs_hbm.at[pl.ds(o0, ROWS_PT)])


def _sc_edge_pass(hh_cat, tab, src, dst, ge, edge_fts):
    mesh = plsc.VectorSubcoreMesh(core_axis_name="c", subcore_axis_name="s",
                                  num_cores=NC, num_subcores=NS)
    zn = jnp.zeros((ROWS_PT, 2 * DOUT), jnp.float32)
    ze = jnp.zeros((ROWS_PT, 2 * EOUT), jnp.float32)
    zs = jnp.zeros((ROWS_PT, 8), jnp.float32)
    run = pl.kernel(
        _sc_body,
        compiler_params=pltpu.CompilerParams(needs_layout_passes=False,
                                             use_tc_tiling_on_sc=False),
        out_type=[
            jax.ShapeDtypeStruct((NC * NP, 2 * DOUT), jnp.float32),
            jax.ShapeDtypeStruct((NC * NP, 2 * EOUT), jnp.float32),
            jax.ShapeDtypeStruct((NC * NP, 8), jnp.float32),
        ],
        mesh=mesh,
        scratch_types=[
            pltpu.VMEM_SHARED((NP, 2 * DOUT), jnp.float32),
            pltpu.VMEM_SHARED((NP, 2 * EOUT), jnp.float32),
            pltpu.VMEM_SHARED((NP, 8), jnp.float32),
            pltpu.VMEM_SHARED((N, 8), jnp.float32),
            pltpu.VMEM((C,), jnp.int32),
            pltpu.VMEM((C,), jnp.int32),
            pltpu.VMEM((C,), jnp.int32),
            pltpu.VMEM((C, 8), jnp.float32),
            pltpu.VMEM((C, 8), jnp.float32),
            pltpu.VMEM((C, H), jnp.float32),
            pltpu.VMEM((C, 2 * DOUT), jnp.float32),
            pltpu.VMEM((C, EIN), jnp.float32),
            pltpu.VMEM((C, 2 * DOUT), jnp.float32),
            pltpu.VMEM((C, 2 * EOUT), jnp.float32),
            pltpu.VMEM((C, 8), jnp.float32),
            pltpu.SemaphoreType.DMA,
            pltpu.SemaphoreType.DMA,
            pltpu.SemaphoreType.DMA,
            pltpu.SemaphoreType.DMA,
            pltpu.SemaphoreType.DMA,
            pltpu.SemaphoreType.DMA,
            pltpu.SemaphoreType.DMA,
        ],
    )
    return run(hh_cat, tab, src, dst, ge, edge_fts, zn, ze, zs)


def _weights_body(stat_ref, o_ref):
    s = stat_ref[...]
    nvars = []
    evars = []
    for c in range(NC):
        for j in range(2):
            den = s[c, :, 4 * j]
            q = s[c, :, 4 * j + 1]
            eden = s[c, :, 4 * j + 2]
            eq = s[c, :, 4 * j + 3]
            mean_n = jnp.sum(den / (den + EPS)) / E
            ex2_n = jnp.sum(q / (den + EPS) ** 2) / E
            mean_e = jnp.sum(eden / (eden + EPS)) / E
            ex2_e = jnp.sum(eq / (eden + EPS) ** 2) / E
            nvars.append(ex2_n - mean_n * mean_n)
            evars.append(ex2_e - mean_e * mean_e)
    nv = jnp.exp(jnp.clip(jnp.stack(nvars), -2.0, 2.0))
    nv = nv / jnp.sum(nv)
    ev = jnp.exp(jnp.clip(jnp.stack(evars), -2.0, 2.0))
    ev = ev / jnp.sum(ev)
    o_ref[...] = jnp.stack([nv, ev])[None]


def _head_weights(stat):
    return pl.pallas_call(
        _weights_body,
        out_shape=jax.ShapeDtypeStruct((1, NC, H), jnp.float32),
    )(stat)


def _finalize_body(accn_ref, acce_ref, stat_ref, w_ref, we_ref, o_ref):
    parts_n = []
    parts_e = []
    for c in range(NC):
        for j in range(2):
            h = 2 * c + j
            den = stat_ref[c, :, 4 * j]
            eden = stat_ref[c, :, 4 * j + 2]
            num = accn_ref[c, :, j * DOUT:(j + 1) * DOUT]
            enum = acce_ref[c, :, j * EOUT:(j + 1) * EOUT]
            nscale = w_ref[0, 0:1, h:h + 1]
            escale = w_ref[0, 1:2, h:h + 1]
            node_out = num / (den[:, None] + EPS) * nscale
            tmp = enum / (eden[:, None] + EPS)
            edge_out = jnp.dot(tmp, we_ref[h],
                               preferred_element_type=jnp.float32) * escale
            parts_n.append(node_out)
            parts_e.append(edge_out)
    o_ref[...] = jnp.concatenate(parts_n + parts_e, axis=1)


def _finalize(accn, acce, stat, w, we):
    nb = N // 10
    return pl.pallas_call(
        _finalize_body,
        grid=(10,),
        in_specs=[
            pl.BlockSpec((NC, nb, 2 * DOUT), lambda i: (0, i, 0)),
            pl.BlockSpec((NC, nb, 2 * EOUT), lambda i: (0, i, 0)),
            pl.BlockSpec((NC, nb, 8), lambda i: (0, i, 0)),
            pl.BlockSpec((1, NC, H), lambda i: (0, 0, 0)),
            pl.BlockSpec((H, EOUT, EOUT), lambda i: (0, 0, 0)),
        ],
        out_specs=pl.BlockSpec((nb, H * (DOUT + EOUT)), lambda i: (i, 0)),
        out_shape=jax.ShapeDtypeStruct((N, H * (DOUT + EOUT)), jnp.float32),
    )(accn, acce, stat, w, we)


def kernel(node_fts, edge_fts, edges, Wn, We, a_node, a_edge):
    w_sd = jnp.einsum('hdo,ho->dh', Wn, a_node[:, :DOUT])
    w_ss = jnp.einsum('hdo,ho->dh', Wn, a_node[:, DOUT:])
    w_esd = jnp.einsum('hdo,ho->dh', Wn, a_edge[:, :DOUT])
    w_ess = jnp.einsum('hdo,ho->dh', Wn, a_edge[:, DOUT:2 * DOUT])
    wea = jnp.einsum('heo,ho->eh', We, a_edge[:, 2 * DOUT:])

    wn_flat = jnp.concatenate([Wn[h] for h in range(H)], axis=1)
    tab_cols = jnp.stack(
        [w_sd[:, 0], w_esd[:, 0], w_sd[:, 1], w_esd[:, 1],
         w_ss[:, 0], w_ess[:, 0], w_ss[:, 1], w_ess[:, 1],
         w_sd[:, 2], w_esd[:, 2], w_sd[:, 3], w_esd[:, 3],
         w_ss[:, 2], w_ess[:, 2], w_ss[:, 3], w_ess[:, 3]], axis=1)
    wbig = jnp.concatenate([wn_flat, tab_cols], axis=1)

    big = _prep_node(node_fts, wbig)
    ge = _prep_edge(edge_fts, wea)

    hh_cat = jnp.concatenate([big[:, :64], big[:, 64:128]], axis=0)
    tab = jnp.concatenate([big[:, 128:136], big[:, 136:144]], axis=0)

    src = jnp.asarray(edges[:, 0], jnp.int32)
    dst = jnp.asarray(edges[:, 1], jnp.int32)

    outn, oute, outs = _sc_edge_pass(hh_cat, tab, src, dst, ge, edge_fts)

    accn = outn.reshape(NC, NP, 2 * DOUT)[:, :N]
    acce = oute.reshape(NC, NP, 2 * EOUT)[:, :N]
    stat = outs.reshape(NC, NP, 8)[:, :N]

    w = _head_weights(stat)
    return _finalize(accn, acce, stat, w, We)


# speedup vs baseline: 34.7594x; 1.0283x over previous
import functools

import jax
import jax.numpy as jnp
from jax import lax
from jax.experimental import pallas as pl
from jax.experimental.pallas import tpu as pltpu
from jax.experimental.pallas import tpu_sc as plsc

N = 10000
E = 320000
DIN = 128
DOUT = 32
EIN = 16
EOUT = 16
H = 4
ALPHA = 0.2
EPS = 1e-16

NC = 2
NS = 16
L = 16

C = 80
EPT = E // NS
NCHUNK = EPT // C
NP = 10240
ROWS_PT = NP // NS


def _matmul_body(x_ref, w_ref, o_ref):
    o_ref[...] = jnp.dot(x_ref[...], w_ref[...],
                         preferred_element_type=jnp.float32)


def _prep_node(node_fts, wbig):
    kcols = wbig.shape[1]
    return pl.pallas_call(
        _matmul_body,
        grid=(10,),
        in_specs=[
            pl.BlockSpec((N // 10, DIN), lambda i: (i, 0)),
            pl.BlockSpec((DIN, kcols), lambda i: (0, 0)),
        ],
        out_specs=pl.BlockSpec((N // 10, kcols), lambda i: (i, 0)),
        out_shape=jax.ShapeDtypeStruct((N, kcols), jnp.float32),
    )(node_fts, wbig)


def _prep_edge(edge_fts, wea):
    return pl.pallas_call(
        _matmul_body,
        grid=(64,),
        in_specs=[
            pl.BlockSpec((E // 64, EIN), lambda i: (i, 0)),
            pl.BlockSpec((EIN, H), lambda i: (0, 0)),
        ],
        out_specs=pl.BlockSpec((E // 64, H), lambda i: (i, 0)),
        out_shape=jax.ShapeDtypeStruct((E, H), jnp.float32),
    )(edge_fts, wea)


def _sc_body(hh_hbm, tab_hbm, src_hbm, dst_hbm, ge_hbm, ef_hbm,
             zn_hbm, ze_hbm, zs_hbm,
             outn_hbm, oute_hbm, outs_hbm,
             shn, she, shs, shtab,
             src_v, dst_v, ge_v, ef_v, src_w, dst_w, ge_w, ef_w,
             idx_v, tdr_v, tsr_v, hh_v,
             nrow_v, erow_v, srow_v,
             gsem, dsem, ssem, osem0, osem1, osem2,
             lsem0, lsem1, lsem2, lsem3, msem0, msem1, msem2, msem3):
    cid = lax.axis_index("c")
    sid = lax.axis_index("s")
    coff = cid * N

    @pl.when(sid == 0)
    def _load_tables():
        pltpu.sync_copy(tab_hbm.at[pl.ds(coff, N)], shtab)

    r0 = sid * ROWS_PT
    pltpu.sync_copy(zn_hbm, shn.at[pl.ds(r0, ROWS_PT)])
    pltpu.sync_copy(ze_hbm, she.at[pl.ds(r0, ROWS_PT)])
    pltpu.sync_copy(zs_hbm, shs.at[pl.ds(r0, ROWS_PT)])
    plsc.subcore_barrier()

    base0 = sid * EPT
    lanes = lax.iota(jnp.int32, 16)

    bufs = [
        (src_v, dst_v, ge_v, ef_v, (lsem0, lsem1, lsem2, lsem3)),
        (src_w, dst_w, ge_w, ef_w, (msem0, msem1, msem2, msem3)),
    ]

    def fire_loads(k, b):
        sv, dv, gv, ev, sems = bufs[b]
        base = jnp.minimum(base0 + k * C, E - C)
        pltpu.async_copy(src_hbm.at[pl.ds(base, C)], sv, sems[0])
        pltpu.async_copy(dst_hbm.at[pl.ds(base, C)], dv, sems[1])
        pltpu.async_copy(ge_hbm.at[pl.ds(base, C)], gv, sems[2])
        pltpu.async_copy(ef_hbm.at[pl.ds(base, C)], ev, sems[3])

    def wait_loads(k, b):
        sv, dv, gv, ev, sems = bufs[b]
        base = jnp.minimum(base0 + k * C, E - C)
        pltpu.make_async_copy(src_hbm.at[pl.ds(base, C)], sv, sems[0]).wait()
        pltpu.make_async_copy(dst_hbm.at[pl.ds(base, C)], dv, sems[1]).wait()
        pltpu.make_async_copy(ge_hbm.at[pl.ds(base, C)], gv, sems[2]).wait()
        pltpu.make_async_copy(ef_hbm.at[pl.ds(base, C)], ev, sems[3]).wait()

    def do_chunk(k, b):
        sv, dv, gv, ev, _ = bufs[b]
        wait_loads(k, b)

        for g in range(C // L):
            idx_v[pl.ds(g * L, L)] = sv[pl.ds(g * L, L)] + coff
        gather = pltpu.async_copy(hh_hbm.at[idx_v], hh_v, gsem)
        gat_d = pltpu.async_copy(shtab.at[dv], tdr_v, dsem)
        gat_s = pltpu.async_copy(shtab.at[sv], tsr_v, ssem)

        fire_loads(k + 1, 1 - b)

        gat_d.wait()
        gat_s.wait()

        scores = []
        for g in range(C // L):
            row_i = lanes + (g * L)
            svecs = []
            for j in range(2):
                c0 = jnp.full((16,), 2 * j, jnp.int32)
                sd = plsc.load_gather(tdr_v, [row_i, c0])
                esd = plsc.load_gather(tdr_v, [row_i, c0 + 1])
                ss = plsc.load_gather(tsr_v, [row_i, c0 + 4])
                ess = plsc.load_gather(tsr_v, [row_i, c0 + 5])
                gcol = jnp.full((16,), 2 * cid + j, jnp.int32)
                gej = plsc.load_gather(gv, [row_i, gcol])
                ns = sd + ss
                ns = jnp.where(ns > 0, ns, ALPHA * ns)
                en = jnp.exp(ns)
                es = esd + ess + gej
                es = jnp.where(es > 0, es, ALPHA * es)
                ee = jnp.exp(es)
                svecs += [en, ee]
                sc0 = jnp.full((16,), 4 * j, jnp.int32)
                plsc.store_scatter(srow_v, [row_i, sc0], en)
                plsc.store_scatter(srow_v, [row_i, sc0 + 1], en * en)
                plsc.store_scatter(srow_v, [row_i, sc0 + 2], ee)
                plsc.store_scatter(srow_v, [row_i, sc0 + 3], ee * ee)
            scores.append(svecs)

        gather.wait()

        for g in range(C // L):
            ena, eea, enb, eeb = scores[g]
            for ii in range(L):
                i = g * L + ii
                ea = ena[ii]
                eb = enb[ii]
                fa = eea[ii]
                fb = eeb[ii]
                for j in range(2):
                    nrow_v[i, pl.ds(j * L, L)] = (
                        ea * hh_v[i, pl.ds(j * L, L)])
                    nrow_v[i, pl.ds(DOUT + j * L, L)] = (
                        eb * hh_v[i, pl.ds(DOUT + j * L, L)])
                ef = ev[i, :]
                erow_v[i, pl.ds(0, L)] = fa * ef
                erow_v[i, pl.ds(L, L)] = fb * ef

        s1 = pltpu.async_copy(srow_v, shs.at[dv], osem0, add=True)
        s2 = pltpu.async_copy(nrow_v, shn.at[dv], osem1, add=True)
        s3 = pltpu.async_copy(erow_v, she.at[dv], osem2, add=True)
        s1.wait()
        s2.wait()
        s3.wait()

    fire_loads(0, 0)

    def pair_body(k2, carry):
        do_chunk(2 * k2, 0)
        do_chunk(2 * k2 + 1, 1)
        return carry

    lax.fori_loop(0, NCHUNK // 2, pair_body, 0)
    wait_loads(NCHUNK, 0)

    plsc.subcore_barrier()
    o0 = cid * NP + r0
    pltpu.sync_copy(shn.at[pl.ds(r0, ROWS_PT)], outn_hbm.at[pl.ds(o0, ROWS_PT)])
    pltpu.sync_copy(she.at[pl.ds(r0, ROWS_PT)], oute_hbm.at[pl.ds(o0, ROWS_PT)])
    pltpu.sync_copy(shs.at[pl.ds(r0, ROWS_PT)], outs_hbm.at[pl.ds(o0, ROWS_PT)])


def _sc_edge_pass(hh_cat, tab, src, dst, ge, edge_fts):
    mesh = plsc.VectorSubcoreMesh(core_axis_name="c", subcore_axis_name="s",
                                  num_cores=NC, num_subcores=NS)
    zn = jnp.zeros((ROWS_PT, 2 * DOUT), jnp.float32)
    ze = jnp.zeros((ROWS_PT, 2 * EOUT), jnp.float32)
    zs = jnp.zeros((ROWS_PT, 8), jnp.float32)
    run = pl.kernel(
        _sc_body,
        compiler_params=pltpu.CompilerParams(needs_layout_passes=False,
                                             use_tc_tiling_on_sc=False),
        out_type=[
            jax.ShapeDtypeStruct((NC * NP, 2 * DOUT), jnp.float32),
            jax.ShapeDtypeStruct((NC * NP, 2 * EOUT), jnp.float32),
            jax.ShapeDtypeStruct((NC * NP, 8), jnp.float32),
        ],
        mesh=mesh,
        scratch_types=[
            pltpu.VMEM_SHARED((NP, 2 * DOUT), jnp.float32),
            pltpu.VMEM_SHARED((NP, 2 * EOUT), jnp.float32),
            pltpu.VMEM_SHARED((NP, 8), jnp.float32),
            pltpu.VMEM_SHARED((N, 8), jnp.float32),
            pltpu.VMEM((C,), jnp.int32),
            pltpu.VMEM((C,), jnp.int32),
            pltpu.VMEM((C, H), jnp.float32),
            pltpu.VMEM((C, EIN), jnp.float32),
            pltpu.VMEM((C,), jnp.int32),
            pltpu.VMEM((C,), jnp.int32),
            pltpu.VMEM((C, H), jnp.float32),
            pltpu.VMEM((C, EIN), jnp.float32),
            pltpu.VMEM((C,), jnp.int32),
            pltpu.VMEM((C, 8), jnp.float32),
            pltpu.VMEM((C, 8), jnp.float32),
            pltpu.VMEM((C, 2 * DOUT), jnp.float32),
            pltpu.VMEM((C, 2 * DOUT), jnp.float32),
            pltpu.VMEM((C, 2 * EOUT), jnp.float32),
            pltpu.VMEM((C, 8), jnp.float32),
        ] + [pltpu.SemaphoreType.DMA] * 14,
    )
    return run(hh_cat, tab, src, dst, ge, edge_fts, zn, ze, zs)


def _weights_body(stat_ref, o_ref):
    s = stat_ref[...]
    nvars = []
    evars = []
    for c in range(NC):
        for j in range(2):
            den = s[c, :, 4 * j]
            q = s[c, :, 4 * j + 1]
            eden = s[c, :, 4 * j + 2]
            eq = s[c, :, 4 * j + 3]
            mean_n = jnp.sum(den / (den + EPS)) / E
            ex2_n = jnp.sum(q / (den + EPS) ** 2) / E
            mean_e = jnp.sum(eden / (eden + EPS)) / E
            ex2_e = jnp.sum(eq / (eden + EPS) ** 2) / E
            nvars.append(ex2_n - mean_n * mean_n)
            evars.append(ex2_e - mean_e * mean_e)
    nv = jnp.exp(jnp.clip(jnp.stack(nvars), -2.0, 2.0))
    nv = nv / jnp.sum(nv)
    ev = jnp.exp(jnp.clip(jnp.stack(evars), -2.0, 2.0))
    ev = ev / jnp.sum(ev)
    o_ref[...] = jnp.stack([nv, ev])[None]


def _head_weights(stat):
    return pl.pallas_call(
        _weights_body,
        out_shape=jax.ShapeDtypeStruct((1, NC, H), jnp.float32),
    )(stat)


def _finalize_body(accn_ref, acce_ref, stat_ref, w_ref, we_ref, o_ref):
    parts_n = []
    parts_e = []
    for c in range(NC):
        for j in range(2):
            h = 2 * c + j
            den = stat_ref[c, :, 4 * j]
            eden = stat_ref[c, :, 4 * j + 2]
            num = accn_ref[c, :, j * DOUT:(j + 1) * DOUT]
            enum = acce_ref[c, :, j * EOUT:(j + 1) * EOUT]
            nscale = w_ref[0, 0:1, h:h + 1]
            escale = w_ref[0, 1:2, h:h + 1]
            node_out = num / (den[:, None] + EPS) * nscale
            tmp = enum / (eden[:, None] + EPS)
            edge_out = jnp.dot(tmp, we_ref[h],
                               preferred_element_type=jnp.float32) * escale
            parts_n.append(node_out)
            parts_e.append(edge_out)
    o_ref[...] = jnp.concatenate(parts_n + parts_e, axis=1)


def _finalize(accn, acce, stat, w, we):
    nb = N // 10
    return pl.pallas_call(
        _finalize_body,
        grid=(10,),
        in_specs=[
            pl.BlockSpec((NC, nb, 2 * DOUT), lambda i: (0, i, 0)),
            pl.BlockSpec((NC, nb, 2 * EOUT), lambda i: (0, i, 0)),
            pl.BlockSpec((NC, nb, 8), lambda i: (0, i, 0)),
            pl.BlockSpec((1, NC, H), lambda i: (0, 0, 0)),
            pl.BlockSpec((H, EOUT, EOUT), lambda i: (0, 0, 0)),
        ],
        out_specs=pl.BlockSpec((nb, H * (DOUT + EOUT)), lambda i: (i, 0)),
        out_shape=jax.ShapeDtypeStruct((N, H * (DOUT + EOUT)), jnp.float32),
    )(accn, acce, stat, w, we)


def kernel(node_fts, edge_fts, edges, Wn, We, a_node, a_edge):
    w_sd = jnp.einsum('hdo,ho->dh', Wn, a_node[:, :DOUT])
    w_ss = jnp.einsum('hdo,ho->dh', Wn, a_node[:, DOUT:])
    w_esd = jnp.einsum('hdo,ho->dh', Wn, a_edge[:, :DOUT])
    w_ess = jnp.einsum('hdo,ho->dh', Wn, a_edge[:, DOUT:2 * DOUT])
    wea = jnp.einsum('heo,ho->eh', We, a_edge[:, 2 * DOUT:])

    wn_flat = jnp.concatenate([Wn[h] for h in range(H)], axis=1)
    tab_cols = jnp.stack(
        [w_sd[:, 0], w_esd[:, 0], w_sd[:, 1], w_esd[:, 1],
         w_ss[:, 0], w_ess[:, 0], w_ss[:, 1], w_ess[:, 1],
         w_sd[:, 2], w_esd[:, 2], w_sd[:, 3], w_esd[:, 3],
         w_ss[:, 2], w_ess[:, 2], w_ss[:, 3], w_ess[:, 3]], axis=1)
    wbig = jnp.concatenate([wn_flat, tab_cols], axis=1)

    big = _prep_node(node_fts, wbig)
    ge = _prep_edge(edge_fts, wea)

    hh_cat = jnp.concatenate([big[:, :64], big[:, 64:128]], axis=0)
    tab = jnp.concatenate([big[:, 128:136], big[:, 136:144]], axis=0)

    src = jnp.asarray(edges[:, 0], jnp.int32)
    dst = jnp.asarray(edges[:, 1], jnp.int32)

    outn, oute, outs = _sc_edge_pass(hh_cat, tab, src, dst, ge, edge_fts)

    accn = outn.reshape(NC, NP, 2 * DOUT)[:, :N]
    acce = oute.reshape(NC, NP, 2 * EOUT)[:, :N]
    stat = outs.reshape(NC, NP, 8)[:, :N]

    w = _head_weights(stat)
    return _finalize(accn, acce, stat, w, We)
